# Initial kernel scaffold; baseline (speedup 1.0000x reference)
import jax
import jax.numpy as jnp
from jax.experimental import pallas as pl


def kernel(x, weight, expert_bias):
    raise NotImplementedError("write your pallas kernel here")


# speedup vs baseline: 1.4792x; 1.4792x over previous
import functools

import jax
import jax.numpy as jnp
from jax.experimental import pallas as pl
from jax.experimental.pallas import tpu as pltpu

N = 16384
DIM = 4096
N_EXPERTS = 64
TOPK = 8
ROUTE_SCALE = 2.5

_BN = 512


def _router_body(x_ref, w_ref, b_ref, wout_ref, iout_ref):
    x = x_ref[...]
    w = w_ref[...]
    logits = jax.lax.dot_general(
        x, w, (((1,), (1,)), ((), ())),
        preferred_element_type=jnp.float32)
    scores = jax.nn.sigmoid(logits)
    biased = scores + b_ref[...]

    iota = jax.lax.broadcasted_iota(jnp.int32, biased.shape, 1)
    vals = biased
    wcols = []
    icols = []
    for _ in range(TOPK):
        m = jnp.max(vals, axis=1, keepdims=True)
        idx = jnp.min(jnp.where(vals == m, iota, N_EXPERTS),
                      axis=1, keepdims=True)
        hit = iota == idx
        wcols.append(jnp.sum(jnp.where(hit, scores, 0.0),
                             axis=1, keepdims=True))
        icols.append(idx)
        vals = jnp.where(hit, -jnp.inf, vals)
    wsel = jnp.concatenate(wcols, axis=1)
    isel = jnp.concatenate(icols, axis=1)
    denom = jnp.sum(wsel, axis=1, keepdims=True) + 1e-8
    wout_ref[...] = wsel / denom * ROUTE_SCALE
    iout_ref[...] = isel


def kernel(x, weight, expert_bias):
    bias2d = expert_bias.reshape(1, N_EXPERTS)
    grid = (N // _BN,)
    wout, iout = pl.pallas_call(
        _router_body,
        grid=grid,
        in_specs=[
            pl.BlockSpec((_BN, DIM), lambda i: (i, 0)),
            pl.BlockSpec((N_EXPERTS, DIM), lambda i: (0, 0)),
            pl.BlockSpec((1, N_EXPERTS), lambda i: (0, 0)),
        ],
        out_specs=[
            pl.BlockSpec((_BN, TOPK), lambda i: (i, 0)),
            pl.BlockSpec((_BN, TOPK), lambda i: (i, 0)),
        ],
        out_shape=[
            jax.ShapeDtypeStruct((N, TOPK), jnp.float32),
            jax.ShapeDtypeStruct((N, TOPK), jnp.int32),
        ],
    )(x, weight, bias2d)
    return wout, iout
